# Initial kernel scaffold; baseline (speedup 1.0000x reference)
import jax
import jax.numpy as jnp
from jax.experimental import pallas as pl


def kernel(x, edge_index, edge_attr, u, batch, W1, b1, W2, b2, W3, b3):
    raise NotImplementedError("write your pallas kernel here")


# speedup vs baseline: 20.2408x; 20.2408x over previous
import jax
import jax.numpy as jnp
from jax import lax
from jax.experimental import pallas as pl
from jax.experimental.pallas import tpu as pltpu
from jax.experimental.pallas import tpu_sc as plsc

NC = 2
NS = 16
SB = 128
K = 8
DP = 8


def _sc_segment_sum(x_pad, ridx, cidx, zeros, n_acc, n_chunks):
    zrows = n_acc // NS
    mesh = plsc.VectorSubcoreMesh(
        core_axis_name="c", subcore_axis_name="s",
        num_cores=NC, num_subcores=NS)

    def body(x_hbm, ridx_hbm, cidx_hbm, zeros_hbm, parts_hbm,
             acc_sh, ridx_v, cidx_v, vals_v, gsem):
        c = lax.axis_index("c")
        s = lax.axis_index("s")
        t = c * NS + s
        pltpu.sync_copy(zeros_hbm, acc_sh.at[pl.ds(s * zrows, zrows)])
        plsc.subcore_barrier()

        def chunk(g, carry):
            sb = (t * n_chunks + g) * K
            pltpu.sync_copy(ridx_hbm.at[pl.ds(sb, K)], ridx_v)
            pltpu.sync_copy(cidx_hbm.at[pl.ds(sb, K)], cidx_v)
            for j in range(K):
                pltpu.async_copy(x_hbm.at[ridx_v.at[j]], vals_v.at[j],
                                 gsem).wait()
                pltpu.sync_copy(vals_v.at[j], acc_sh.at[cidx_v.at[j]],
                                add=True)
            return carry

        lax.fori_loop(0, n_chunks, chunk, 0)
        plsc.subcore_barrier()
        pltpu.sync_copy(acc_sh.at[pl.ds(s * zrows, zrows)],
                        parts_hbm.at[c, pl.ds(s * zrows, zrows)])

    f = pl.kernel(
        body,
        out_type=jax.ShapeDtypeStruct((NC, n_acc, DP), jnp.float32),
        mesh=mesh,
        scratch_types=[
            pltpu.VMEM_SHARED((n_acc, DP), jnp.float32),
            pltpu.VMEM((K, SB), jnp.int32),
            pltpu.VMEM((K, SB), jnp.int32),
            pltpu.VMEM((K, SB, DP), jnp.float32),
            pltpu.SemaphoreType.DMA,
        ],
        compiler_params=pltpu.CompilerParams(use_tc_tiling_on_sc=False),
    )
    return f(x_pad, ridx, cidx, zeros)


def _count_nonzero(x):
    n, d = x.shape
    blk = 20000

    def body(x_ref, o_ref):
        @pl.when(pl.program_id(0) == 0)
        def _():
            o_ref[0, 0] = 0.0
        nz = jnp.any(x_ref[...] != 0.0, axis=1)
        o_ref[0, 0] += jnp.sum(nz.astype(jnp.float32))

    return pl.pallas_call(
        body,
        grid=(n // blk,),
        in_specs=[pl.BlockSpec((blk, d), lambda i: (i, 0))],
        out_specs=pl.BlockSpec(memory_space=pltpu.SMEM),
        out_shape=jax.ShapeDtypeStruct((1, 1), jnp.float32),
    )(x)


def _mlp(x, parts, nnz, W1, b1, W2, b2, W3, b3):
    n, d = x.shape
    h1 = W1.shape[1]
    h2 = W2.shape[1]
    do = W3.shape[1]
    dp = parts.shape[2]
    blk = 2000
    nb = n // blk

    def body(nnz_ref, x_ref, p_ref, w1_ref, b1_ref, w2_ref, b2_ref,
             w3_ref, b3_ref, o_ref):
        denom = jnp.maximum(nnz_ref[0, 0], 1.0)
        s = (p_ref[0] + p_ref[1])[:, :d] / denom
        xb = x_ref[...]
        w1 = w1_ref[...]
        h = (jnp.dot(xb, w1[:d], preferred_element_type=jnp.float32)
             + jnp.dot(s, w1[d:], preferred_element_type=jnp.float32)
             + b1_ref[...])
        h = jnp.maximum(h, 0.0)
        h = jnp.dot(h, w2_ref[...], preferred_element_type=jnp.float32) + b2_ref[...]
        h = jnp.maximum(h, 0.0)
        o_ref[...] = (jnp.dot(h, w3_ref[...], preferred_element_type=jnp.float32)
                      + b3_ref[...])

    return pl.pallas_call(
        body,
        grid=(nb,),
        in_specs=[
            pl.BlockSpec(memory_space=pltpu.SMEM),
            pl.BlockSpec((blk, d), lambda i: (i, 0)),
            pl.BlockSpec((NC, blk, dp), lambda i: (0, i, 0)),
            pl.BlockSpec((2 * d, h1), lambda i: (0, 0)),
            pl.BlockSpec((1, h1), lambda i: (0, 0)),
            pl.BlockSpec((h1, h2), lambda i: (0, 0)),
            pl.BlockSpec((1, h2), lambda i: (0, 0)),
            pl.BlockSpec((h2, do), lambda i: (0, 0)),
            pl.BlockSpec((1, do), lambda i: (0, 0)),
        ],
        out_specs=pl.BlockSpec((blk, do), lambda i: (i, 0)),
        out_shape=jax.ShapeDtypeStruct((n, do), jnp.float32),
    )(nnz, x, parts, W1, b1.reshape(1, -1), W2, b2.reshape(1, -1),
      W3, b3.reshape(1, -1))


def kernel(x, edge_index, edge_attr, u, batch, W1, b1, W2, b2, W3, b3):
    n, d = x.shape
    e = edge_index.shape[1]
    per_round = NC * NS * K * SB
    n_chunks = -(-e // per_round)
    ep = n_chunks * per_round
    row = edge_index[0]
    col = edge_index[1]
    if ep != e:
        pad = ep - e
        row = jnp.concatenate([row, jnp.zeros((pad,), jnp.int32)])
        col = jnp.concatenate([col, jnp.full((pad,), n, jnp.int32)])
    ridx = row.reshape(ep // SB, SB)
    cidx = col.reshape(ep // SB, SB)
    x_pad = jnp.pad(x, ((0, 0), (0, DP - d)))
    n_acc = 8 * NS * (-(-(n + 1) // (8 * NS)))
    zeros = jnp.zeros((n_acc // NS, DP), jnp.float32)
    parts = _sc_segment_sum(x_pad, ridx, cidx, zeros, n_acc, n_chunks)
    nnz = _count_nonzero(x)
    return _mlp(x, parts, nnz, W1, b1, W2, b2, W3, b3)


# speedup vs baseline: 39.0662x; 1.9301x over previous
import jax
import jax.numpy as jnp
from jax import lax
from jax.experimental import pallas as pl
from jax.experimental.pallas import tpu as pltpu
from jax.experimental.pallas import tpu_sc as plsc

NC = 2
NS = 16
SB = 128
K = 8
DP = 8


def _sc_segment_sum(x_pad, ridx, cidx, zeros, n_acc, n_chunks):
    zrows = n_acc // NS
    mesh = plsc.VectorSubcoreMesh(
        core_axis_name="c", subcore_axis_name="s",
        num_cores=NC, num_subcores=NS)

    npairs = n_chunks // 2

    def body(x_hbm, ridx_hbm, cidx_hbm, zeros_hbm, parts_hbm,
             acc_sh, ridx_v, cidx_v, vals_v,
             si0, si1, sg0, sg1, ss0, ss1):
        c = lax.axis_index("c")
        s = lax.axis_index("s")
        t = c * NS + s
        si = (si0, si1)
        sg = (sg0, sg1)
        ss = (ss0, ss1)
        pltpu.sync_copy(zeros_hbm, acc_sh.at[pl.ds(s * zrows, zrows)])
        plsc.subcore_barrier()

        def fire_idx(g, b):
            sb = (t * n_chunks + g) * K
            pltpu.async_copy(ridx_hbm.at[pl.ds(sb, K)], ridx_v.at[b], si[b])
            pltpu.async_copy(cidx_hbm.at[pl.ds(sb, K)], cidx_v.at[b], si[b])

        def wait_idx(b):
            pltpu.make_async_copy(ridx_hbm.at[pl.ds(0, K)], ridx_v.at[b],
                                  si[b]).wait()
            pltpu.make_async_copy(cidx_hbm.at[pl.ds(0, K)], cidx_v.at[b],
                                  si[b]).wait()

        def fire_gathers(b):
            for j in range(K):
                pltpu.async_copy(x_hbm.at[ridx_v.at[b, j]], vals_v.at[b, j],
                                 sg[b])

        def wait_gathers(b):
            for j in range(K):
                pltpu.make_async_copy(x_hbm.at[ridx_v.at[b, j]],
                                      vals_v.at[b, j], sg[b]).wait()

        def fire_scatters(b):
            for j in range(K):
                pltpu.async_copy(vals_v.at[b, j], acc_sh.at[cidx_v.at[b, j]],
                                 ss[b], add=True)

        def wait_scatters(b):
            for j in range(K):
                pltpu.make_async_copy(vals_v.at[b, j],
                                      acc_sh.at[cidx_v.at[b, j]],
                                      ss[b]).wait()

        def do_pair(i, first, last):
            wait_idx(0)
            if not first:
                wait_scatters(0)
            fire_gathers(0)
            wait_idx(1)
            if not first:
                wait_scatters(1)
            wait_gathers(0)
            fire_scatters(0)
            fire_gathers(1)
            if not last:
                fire_idx(2 * i + 2, 0)
            wait_gathers(1)
            fire_scatters(1)
            if not last:
                fire_idx(2 * i + 3, 1)

        fire_idx(0, 0)
        fire_idx(1, 1)
        do_pair(0, True, npairs == 1)
        if npairs > 2:
            def pair_body(i, carry):
                do_pair(i, False, False)
                return carry
            lax.fori_loop(1, npairs - 1, pair_body, 0)
        if npairs > 1:
            do_pair(npairs - 1, False, True)
        wait_scatters(0)
        wait_scatters(1)

        plsc.subcore_barrier()
        pltpu.sync_copy(acc_sh.at[pl.ds(s * zrows, zrows)],
                        parts_hbm.at[c, pl.ds(s * zrows, zrows)])

    f = pl.kernel(
        body,
        out_type=jax.ShapeDtypeStruct((NC, n_acc, DP), jnp.float32),
        mesh=mesh,
        scratch_types=[
            pltpu.VMEM_SHARED((n_acc, DP), jnp.float32),
            pltpu.VMEM((2, K, SB), jnp.int32),
            pltpu.VMEM((2, K, SB), jnp.int32),
            pltpu.VMEM((2, K, SB, DP), jnp.float32),
            pltpu.SemaphoreType.DMA,
            pltpu.SemaphoreType.DMA,
            pltpu.SemaphoreType.DMA,
            pltpu.SemaphoreType.DMA,
            pltpu.SemaphoreType.DMA,
            pltpu.SemaphoreType.DMA,
        ],
        compiler_params=pltpu.CompilerParams(use_tc_tiling_on_sc=False),
    )
    return f(x_pad, ridx, cidx, zeros)


def _count_nonzero(x):
    n, d = x.shape
    blk = 20000

    def body(x_ref, o_ref):
        @pl.when(pl.program_id(0) == 0)
        def _():
            o_ref[0, 0] = 0.0
        nz = jnp.any(x_ref[...] != 0.0, axis=1)
        o_ref[0, 0] += jnp.sum(nz.astype(jnp.float32))

    return pl.pallas_call(
        body,
        grid=(n // blk,),
        in_specs=[pl.BlockSpec((blk, d), lambda i: (i, 0))],
        out_specs=pl.BlockSpec(memory_space=pltpu.SMEM),
        out_shape=jax.ShapeDtypeStruct((1, 1), jnp.float32),
    )(x)


def _mlp(x, parts, nnz, W1, b1, W2, b2, W3, b3):
    n, d = x.shape
    h1 = W1.shape[1]
    h2 = W2.shape[1]
    do = W3.shape[1]
    dp = parts.shape[2]
    blk = 2000
    nb = n // blk

    def body(nnz_ref, x_ref, p_ref, w1_ref, b1_ref, w2_ref, b2_ref,
             w3_ref, b3_ref, o_ref):
        denom = jnp.maximum(nnz_ref[0, 0], 1.0)
        s = (p_ref[0] + p_ref[1])[:, :d] / denom
        xb = x_ref[...]
        w1 = w1_ref[...]
        h = (jnp.dot(xb, w1[:d], preferred_element_type=jnp.float32)
             + jnp.dot(s, w1[d:], preferred_element_type=jnp.float32)
             + b1_ref[...])
        h = jnp.maximum(h, 0.0)
        h = jnp.dot(h, w2_ref[...], preferred_element_type=jnp.float32) + b2_ref[...]
        h = jnp.maximum(h, 0.0)
        o_ref[...] = (jnp.dot(h, w3_ref[...], preferred_element_type=jnp.float32)
                      + b3_ref[...])

    return pl.pallas_call(
        body,
        grid=(nb,),
        in_specs=[
            pl.BlockSpec(memory_space=pltpu.SMEM),
            pl.BlockSpec((blk, d), lambda i: (i, 0)),
            pl.BlockSpec((NC, blk, dp), lambda i: (0, i, 0)),
            pl.BlockSpec((2 * d, h1), lambda i: (0, 0)),
            pl.BlockSpec((1, h1), lambda i: (0, 0)),
            pl.BlockSpec((h1, h2), lambda i: (0, 0)),
            pl.BlockSpec((1, h2), lambda i: (0, 0)),
            pl.BlockSpec((h2, do), lambda i: (0, 0)),
            pl.BlockSpec((1, do), lambda i: (0, 0)),
        ],
        out_specs=pl.BlockSpec((blk, do), lambda i: (i, 0)),
        out_shape=jax.ShapeDtypeStruct((n, do), jnp.float32),
    )(nnz, x, parts, W1, b1.reshape(1, -1), W2, b2.reshape(1, -1),
      W3, b3.reshape(1, -1))


def kernel(x, edge_index, edge_attr, u, batch, W1, b1, W2, b2, W3, b3):
    n, d = x.shape
    e = edge_index.shape[1]
    per_round = NC * NS * K * SB
    n_chunks = -(-e // per_round)
    n_chunks += n_chunks % 2
    ep = n_chunks * per_round
    row = edge_index[0]
    col = edge_index[1]
    if ep != e:
        pad = ep - e
        row = jnp.concatenate([row, jnp.zeros((pad,), jnp.int32)])
        col = jnp.concatenate([col, jnp.full((pad,), n, jnp.int32)])
    ridx = row.reshape(ep // SB, SB)
    cidx = col.reshape(ep // SB, SB)
    x_pad = jnp.pad(x, ((0, 0), (0, DP - d)))
    n_acc = 8 * NS * (-(-(n + 1) // (8 * NS)))
    zeros = jnp.zeros((n_acc // NS, DP), jnp.float32)
    parts = _sc_segment_sum(x_pad, ridx, cidx, zeros, n_acc, n_chunks)
    nnz = _count_nonzero(x)
    return _mlp(x, parts, nnz, W1, b1, W2, b2, W3, b3)
